# baseline (speedup 1.0000x reference)
import functools
import math

import jax
import jax.numpy as jnp
from jax import lax
from jax.experimental import pallas as pl
from jax.experimental.pallas import tpu as pltpu
from jax.experimental.pallas import tpu_sc as plsc

_SC_CORES = 2
_SC_SUBCORES = 16
_LANES = 16


def kernel(hidden_states, Wq, bq, Wk, bk, Wv, bv, Wo, bo,
           memory_keys, memory_values, memory_age):
    batch, seq, hidden = hidden_states.shape
    heads, msize, _ = memory_keys.shape
    f32 = jnp.float32
    i32 = jnp.int32
    scale = 1.0 / math.sqrt(hidden)

    hs = hidden_states.reshape(batch, hidden)
    K2 = memory_keys.reshape(msize, hidden)
    V2 = memory_values.reshape(msize, hidden)
    age2 = memory_age.reshape(1, msize)

    BM = 4096
    NB = msize // BM
    dn_nt = (((1,), (1,)), ((), ()))
    dn_nn = (((1,), (0,)), ((), ()))

    NW = _SC_CORES * _SC_SUBCORES
    CH = msize // NW
    NSL = CH // _LANES

    def _kstream(hs_ref, wq_ref, bq_ref, wk_ref, bk_ref, wv_ref, bv_ref,
                 k_ref,
                 kp_ref, vp_ref, s_ref, nk_ref, m_ref, l_ref, ms_ref,
                 q_scr, m_scr, l_scr):
        i = pl.program_id(0)

        @pl.when(i == 0)
        def _():
            x = hs_ref[...]
            q_scr[...] = jax.lax.dot_general(
                x, wq_ref[...], dn_nt, preferred_element_type=f32) + bq_ref[...]
            kp_ref[...] = jax.lax.dot_general(
                x, wk_ref[...], dn_nt, preferred_element_type=f32) + bk_ref[...]
            vp_ref[...] = jax.lax.dot_general(
                x, wv_ref[...], dn_nt, preferred_element_type=f32) + bv_ref[...]
            m_scr[...] = jnp.full(m_scr.shape, -jnp.inf, f32)
            l_scr[...] = jnp.zeros(l_scr.shape, f32)

        kblk = k_ref[...]
        s = jax.lax.dot_general(q_scr[...], kblk, dn_nt,
                                preferred_element_type=f32) * scale
        s_ref[...] = s
        nk_ref[...] = kblk

        bmax = jnp.max(s, axis=1, keepdims=True)
        m_old = m_scr[...]
        m_new = jnp.maximum(m_old, bmax)
        l_new = (l_scr[...] * jnp.exp(m_old - m_new)
                 + jnp.sum(jnp.exp(s - bmax), axis=1, keepdims=True)
                 * jnp.exp(bmax - m_new))
        m_scr[...] = m_new
        l_scr[...] = l_new

        @pl.when(i == NB - 1)
        def _():
            m_ref[...] = m_new
            l_ref[...] = l_new
            ms_ref[...] = jnp.full(ms_ref.shape, jnp.mean(m_new[:, 0:1]), f32)

    kproj, vproj, scores, new_keys2, mrow, lrow, msarr = pl.pallas_call(
        _kstream,
        grid=(NB,),
        in_specs=[pl.BlockSpec((batch, hidden), lambda i: (0, 0)),
                  pl.BlockSpec((hidden, hidden), lambda i: (0, 0)),
                  pl.BlockSpec((1, hidden), lambda i: (0, 0)),
                  pl.BlockSpec((hidden, hidden), lambda i: (0, 0)),
                  pl.BlockSpec((1, hidden), lambda i: (0, 0)),
                  pl.BlockSpec((hidden, hidden), lambda i: (0, 0)),
                  pl.BlockSpec((1, hidden), lambda i: (0, 0)),
                  pl.BlockSpec((BM, hidden), lambda i: (i, 0))],
        out_specs=[pl.BlockSpec((batch, hidden), lambda i: (0, 0)),
                   pl.BlockSpec((batch, hidden), lambda i: (0, 0)),
                   pl.BlockSpec((batch, BM), lambda i: (0, i)),
                   pl.BlockSpec((BM, hidden), lambda i: (i, 0)),
                   pl.BlockSpec((batch, 128), lambda i: (0, 0)),
                   pl.BlockSpec((batch, 128), lambda i: (0, 0)),
                   pl.BlockSpec((batch, 128), lambda i: (0, 0))],
        out_shape=[jax.ShapeDtypeStruct((batch, hidden), f32),
                   jax.ShapeDtypeStruct((batch, hidden), f32),
                   jax.ShapeDtypeStruct((batch, msize), f32),
                   jax.ShapeDtypeStruct((msize, hidden), f32),
                   jax.ShapeDtypeStruct((batch, 128), f32),
                   jax.ShapeDtypeStruct((batch, 128), f32),
                   jax.ShapeDtypeStruct((batch, 128), f32)],
        scratch_shapes=[pltpu.VMEM((batch, hidden), f32),
                        pltpu.VMEM((batch, 128), f32),
                        pltpu.VMEM((batch, 128), f32)],
    )(hs, Wq, bq.reshape(1, hidden), Wk, bk.reshape(1, hidden),
      Wv, bv.reshape(1, hidden), K2)

    mesh = plsc.VectorSubcoreMesh(core_axis_name="c", subcore_axis_name="s",
                                  num_cores=_SC_CORES,
                                  num_subcores=_SC_SUBCORES)

    @functools.partial(
        pl.kernel, mesh=mesh,
        out_type=[jax.ShapeDtypeStruct((1, msize), i32),
                  jax.ShapeDtypeStruct((1, msize), f32),
                  jax.ShapeDtypeStruct((NW, 128), f32),
                  jax.ShapeDtypeStruct((NW, 128), i32),
                  jax.ShapeDtypeStruct((NW, 128), f32),
                  jax.ShapeDtypeStruct((NW, 128), f32)],
        scratch_types=[pltpu.VMEM((batch, CH), f32),
                       pltpu.VMEM((batch, 128), f32),
                       pltpu.VMEM((batch, 128), f32),
                       pltpu.VMEM((CH,), f32),
                       pltpu.VMEM((CH,), f32),
                       pltpu.VMEM((CH,), i32),
                       pltpu.VMEM((_LANES,), f32),
                       pltpu.VMEM((_LANES,), i32),
                       pltpu.VMEM((_LANES,), f32),
                       pltpu.VMEM((_LANES,), f32),
                       pltpu.VMEM((_LANES,), i32)],
    )
    def _sc_select(s_hbm, m_hbm, l_hbm, age_hbm,
                   ac_hbm, na_hbm, rmax_hbm, rpos_hbm, rna_hbm, cnt_hbm,
                   sv, mv, lv, agev, nav, acv, rmax, rpos, rna, rcnt, posv):
        wid = lax.axis_index("s") * _SC_CORES + lax.axis_index("c")
        base = wid * CH
        for b in range(batch):
            pltpu.sync_copy(s_hbm.at[b, pl.ds(base, CH)], sv.at[b])
        pltpu.sync_copy(m_hbm, mv)
        pltpu.sync_copy(l_hbm, lv)
        pltpu.sync_copy(age_hbm.at[0, pl.ds(base, CH)], agev)

        rmax[...] = jnp.full((_LANES,), -jnp.inf, f32)
        rpos[...] = jnp.zeros((_LANES,), i32)
        rna[...] = jnp.zeros((_LANES,), f32)
        rcnt[...] = jnp.zeros((_LANES,), f32)
        posv[...] = lax.iota(i32, _LANES)
        mvec = [mv[b, pl.ds(0, _LANES)] for b in range(batch)]
        lvec = [lv[b, pl.ds(0, _LANES)] for b in range(batch)]

        @pl.loop(0, NSL)
        def _(j):
            sl = pl.ds(j * _LANES, _LANES)
            imp = jnp.zeros((_LANES,), f32)
            ac = jnp.zeros((_LANES,), i32)
            for b in range(batch):
                p = jnp.exp(sv[b, sl] - mvec[b]) / lvec[b]
                imp = imp + p
                ac = ac + jnp.where(p > 0.01, 1, 0).astype(i32)
            na = agev[sl] + 1.0
            nav[sl] = na
            acv[sl] = ac
            t = na + (1.0 - imp)
            pv = posv[...]
            upd = t > rmax[...]
            rmax[...] = jnp.where(upd, t, rmax[...])
            rpos[...] = jnp.where(upd, pv, rpos[...])
            rna[...] = jnp.where(upd, na, rna[...])
            rcnt[...] = rcnt[...] + jnp.where(na > 0.0, 1.0, 0.0)
            posv[...] = pv + _LANES

        pltpu.sync_copy(acv, ac_hbm.at[0, pl.ds(base, CH)])
        pltpu.sync_copy(nav, na_hbm.at[0, pl.ds(base, CH)])
        pltpu.sync_copy(rmax, rmax_hbm.at[wid, pl.ds(0, _LANES)])
        pltpu.sync_copy(rpos, rpos_hbm.at[wid, pl.ds(0, _LANES)])
        pltpu.sync_copy(rna, rna_hbm.at[wid, pl.ds(0, _LANES)])
        pltpu.sync_copy(rcnt, cnt_hbm.at[wid, pl.ds(0, _LANES)])

    ac_row, na_row, rmax_a, rpos_a, rna_a, cnt_a = _sc_select(
        scores, mrow, lrow, age2)

    def _vstream(s_ref, m_ref, l_ref, v_ref, wo_ref, bo_ref,
                 nv_ref, y_ref, o_scr):
        i = pl.program_id(0)

        @pl.when(i == 0)
        def _():
            o_scr[...] = jnp.zeros(o_scr.shape, f32)

        m = m_ref[:, 0:1]
        l = l_ref[:, 0:1]
        p = jnp.exp(s_ref[...] - m) / l
        v = v_ref[...]
        nv_ref[...] = v
        o_scr[...] += jax.lax.dot_general(p, v, dn_nn,
                                          preferred_element_type=f32)

        @pl.when(i == NB - 1)
        def _():
            y_ref[...] = jax.lax.dot_general(
                o_scr[...], wo_ref[...], dn_nt,
                preferred_element_type=f32) + bo_ref[...]

    new_values2, out = pl.pallas_call(
        _vstream,
        grid=(NB,),
        in_specs=[pl.BlockSpec((batch, BM), lambda i: (0, i)),
                  pl.BlockSpec((batch, 128), lambda i: (0, 0)),
                  pl.BlockSpec((batch, 128), lambda i: (0, 0)),
                  pl.BlockSpec((BM, hidden), lambda i: (i, 0)),
                  pl.BlockSpec((hidden, hidden), lambda i: (0, 0)),
                  pl.BlockSpec((1, hidden), lambda i: (0, 0))],
        out_specs=[pl.BlockSpec((BM, hidden), lambda i: (i, 0)),
                   pl.BlockSpec((batch, hidden), lambda i: (0, 0))],
        out_shape=[jax.ShapeDtypeStruct((msize, hidden), f32),
                   jax.ShapeDtypeStruct((batch, hidden), f32)],
        scratch_shapes=[pltpu.VMEM((batch, hidden), f32)],
    )(scores, mrow, lrow, V2, Wo, bo.reshape(1, hidden))

    def _select(rmax_ref, rpos_ref, rna_ref, cnt_ref, idx_ref, usage_ref):
        sub = rmax_ref[:, 0:_LANES]
        pos = rpos_ref[:, 0:_LANES]
        rna = rna_ref[:, 0:_LANES]
        cnt = cnt_ref[:, 0:_LANES]
        rowi = jax.lax.broadcasted_iota(i32, sub.shape, 0)
        pabs = rowi * CH + pos
        gmax = jnp.max(sub)
        cand = jnp.where(sub == gmax, pabs, msize)
        gidx = jnp.min(cand)
        na_at = jnp.sum(jnp.where((sub == gmax) & (pabs == gidx), rna, 0.0))
        npos = jnp.sum(cnt)
        usage = (npos - (na_at > 0.0).astype(f32)) / msize
        idx_ref[...] = jnp.full(idx_ref.shape, gidx, i32)
        usage_ref[...] = jnp.full(usage_ref.shape, usage, f32)

    idx_out, usage_out = pl.pallas_call(
        _select,
        out_shape=[jax.ShapeDtypeStruct((1, 128), i32),
                   jax.ShapeDtypeStruct((1, 128), f32)],
    )(rmax_a, rpos_a, rna_a, cnt_a)

    idx1 = idx_out[0, 0:1]
    updk = kproj[0:1]
    updv = vproj[0:1]

    def _scatter(idx_ref, updk_ref, updv_ref, kin_ref, vin_ref, ain_ref,
                 kout_ref, vout_ref, aout_ref):
        row = idx_ref[0] % 8
        lane = idx_ref[0] % 128
        rowv = jax.lax.broadcasted_iota(i32, kin_ref.shape, 0)
        kout_ref[...] = jnp.where(rowv == row, updk_ref[...], kin_ref[...])
        vout_ref[...] = jnp.where(rowv == row, updv_ref[...], vin_ref[...])
        colv = jax.lax.broadcasted_iota(i32, ain_ref.shape, 1)
        aout_ref[...] = jnp.where(colv == lane, 0.0, ain_ref[...])

    grid_spec = pltpu.PrefetchScalarGridSpec(
        num_scalar_prefetch=1,
        grid=(1,),
        in_specs=[
            pl.BlockSpec((1, hidden), lambda i, idx: (0, 0)),
            pl.BlockSpec((1, hidden), lambda i, idx: (0, 0)),
            pl.BlockSpec((8, hidden), lambda i, idx: (idx[0] // 8, 0)),
            pl.BlockSpec((8, hidden), lambda i, idx: (idx[0] // 8, 0)),
            pl.BlockSpec((1, 128), lambda i, idx: (0, idx[0] // 128)),
        ],
        out_specs=[
            pl.BlockSpec((8, hidden), lambda i, idx: (idx[0] // 8, 0)),
            pl.BlockSpec((8, hidden), lambda i, idx: (idx[0] // 8, 0)),
            pl.BlockSpec((1, 128), lambda i, idx: (0, idx[0] // 128)),
        ],
    )
    nk_f, nv_f, na_f = pl.pallas_call(
        _scatter,
        grid_spec=grid_spec,
        out_shape=[jax.ShapeDtypeStruct((msize, hidden), f32),
                   jax.ShapeDtypeStruct((msize, hidden), f32),
                   jax.ShapeDtypeStruct((1, msize), f32)],
        input_output_aliases={3: 0, 4: 1, 5: 2},
    )(idx1, updk, updv, new_keys2, new_values2, na_row)

    output = out.reshape(batch, seq, hidden)
    access_counts = ac_row.reshape(heads, msize)
    max_scores = msarr[0, 0]
    memory_usage = usage_out[0, 0]
    new_keys = nk_f.reshape(heads, msize, hidden)
    new_values = nv_f.reshape(heads, msize, hidden)
    new_age = na_f.reshape(heads, msize)
    return (output, access_counts, max_scores, memory_usage,
            new_keys, new_values, new_age)


# speedup vs baseline: 1.0123x; 1.0123x over previous
import functools
import math

import jax
import jax.numpy as jnp
from jax import lax
from jax.experimental import pallas as pl
from jax.experimental.pallas import tpu as pltpu
from jax.experimental.pallas import tpu_sc as plsc

_SC_CORES = 2
_SC_SUBCORES = 16
_LANES = 16


def kernel(hidden_states, Wq, bq, Wk, bk, Wv, bv, Wo, bo,
           memory_keys, memory_values, memory_age):
    batch, seq, hidden = hidden_states.shape
    heads, msize, _ = memory_keys.shape
    f32 = jnp.float32
    i32 = jnp.int32
    scale = 1.0 / math.sqrt(hidden)

    hs = hidden_states.reshape(batch, hidden)
    K2 = memory_keys.reshape(msize, hidden)
    V2 = memory_values.reshape(msize, hidden)
    age2 = memory_age.reshape(1, msize)

    BM = 4096
    NB = msize // BM
    dn_nt = (((1,), (1,)), ((), ()))
    dn_nn = (((1,), (0,)), ((), ()))

    NW = _SC_CORES * _SC_SUBCORES
    CH = msize // NW
    NSL = CH // _LANES

    def _kstream(hs_ref, wq_ref, bq_ref, wk_ref, bk_ref, wv_ref, bv_ref,
                 k_ref,
                 kp_ref, vp_ref, s_ref, nk_ref, m_ref, l_ref, ms_ref,
                 q_scr, m_scr, l_scr):
        i = pl.program_id(0)

        @pl.when(i == 0)
        def _():
            x = hs_ref[...]
            q_scr[...] = jax.lax.dot_general(
                x, wq_ref[...], dn_nt, preferred_element_type=f32) + bq_ref[...]
            kp_ref[...] = jax.lax.dot_general(
                x, wk_ref[...], dn_nt, preferred_element_type=f32) + bk_ref[...]
            vp_ref[...] = jax.lax.dot_general(
                x, wv_ref[...], dn_nt, preferred_element_type=f32) + bv_ref[...]
            m_scr[...] = jnp.full(m_scr.shape, -jnp.inf, f32)
            l_scr[...] = jnp.zeros(l_scr.shape, f32)

        kblk = k_ref[...]
        s = jax.lax.dot_general(q_scr[...], kblk, dn_nt,
                                preferred_element_type=f32) * scale
        s_ref[...] = s
        nk_ref[...] = kblk

        bmax = jnp.max(s, axis=1, keepdims=True)
        m_old = m_scr[...]
        m_new = jnp.maximum(m_old, bmax)
        l_new = (l_scr[...] * jnp.exp(m_old - m_new)
                 + jnp.sum(jnp.exp(s - bmax), axis=1, keepdims=True)
                 * jnp.exp(bmax - m_new))
        m_scr[...] = m_new
        l_scr[...] = l_new

        @pl.when(i == NB - 1)
        def _():
            m_ref[...] = m_new
            l_ref[...] = l_new
            ms_ref[...] = jnp.full(ms_ref.shape, jnp.mean(m_new[:, 0:1]), f32)

    kproj, vproj, scores, new_keys2, mrow, lrow, msarr = pl.pallas_call(
        _kstream,
        grid=(NB,),
        in_specs=[pl.BlockSpec((batch, hidden), lambda i: (0, 0)),
                  pl.BlockSpec((hidden, hidden), lambda i: (0, 0)),
                  pl.BlockSpec((1, hidden), lambda i: (0, 0)),
                  pl.BlockSpec((hidden, hidden), lambda i: (0, 0)),
                  pl.BlockSpec((1, hidden), lambda i: (0, 0)),
                  pl.BlockSpec((hidden, hidden), lambda i: (0, 0)),
                  pl.BlockSpec((1, hidden), lambda i: (0, 0)),
                  pl.BlockSpec((BM, hidden), lambda i: (i, 0))],
        out_specs=[pl.BlockSpec((batch, hidden), lambda i: (0, 0)),
                   pl.BlockSpec((batch, hidden), lambda i: (0, 0)),
                   pl.BlockSpec((batch, BM), lambda i: (0, i)),
                   pl.BlockSpec((BM, hidden), lambda i: (i, 0)),
                   pl.BlockSpec((batch, 128), lambda i: (0, 0)),
                   pl.BlockSpec((batch, 128), lambda i: (0, 0)),
                   pl.BlockSpec((batch, 128), lambda i: (0, 0))],
        out_shape=[jax.ShapeDtypeStruct((batch, hidden), f32),
                   jax.ShapeDtypeStruct((batch, hidden), f32),
                   jax.ShapeDtypeStruct((batch, msize), f32),
                   jax.ShapeDtypeStruct((msize, hidden), f32),
                   jax.ShapeDtypeStruct((batch, 128), f32),
                   jax.ShapeDtypeStruct((batch, 128), f32),
                   jax.ShapeDtypeStruct((batch, 128), f32)],
        scratch_shapes=[pltpu.VMEM((batch, hidden), f32),
                        pltpu.VMEM((batch, 128), f32),
                        pltpu.VMEM((batch, 128), f32)],
    )(hs, Wq, bq.reshape(1, hidden), Wk, bk.reshape(1, hidden),
      Wv, bv.reshape(1, hidden), K2)

    mesh = plsc.VectorSubcoreMesh(core_axis_name="c", subcore_axis_name="s",
                                  num_cores=_SC_CORES,
                                  num_subcores=_SC_SUBCORES)

    @functools.partial(
        pl.kernel, mesh=mesh,
        out_type=[jax.ShapeDtypeStruct((1, msize), i32),
                  jax.ShapeDtypeStruct((1, msize), f32),
                  jax.ShapeDtypeStruct((NW, 128), f32),
                  jax.ShapeDtypeStruct((NW, 128), i32),
                  jax.ShapeDtypeStruct((NW, 128), f32),
                  jax.ShapeDtypeStruct((NW, 128), f32)],
        scratch_types=[pltpu.VMEM((batch, CH), f32),
                       pltpu.VMEM((batch, 128), f32),
                       pltpu.VMEM((batch, 128), f32),
                       pltpu.VMEM((CH,), f32),
                       pltpu.VMEM((CH,), f32),
                       pltpu.VMEM((CH,), i32),
                       pltpu.VMEM((CH,), f32),
                       pltpu.VMEM((_LANES,), f32),
                       pltpu.VMEM((_LANES,), i32),
                       pltpu.VMEM((_LANES,), f32),
                       pltpu.VMEM((_LANES,), f32),
                       pltpu.VMEM((_LANES,), i32),
                       pltpu.SemaphoreType.DMA],
    )
    def _sc_select(s_hbm, m_hbm, l_hbm, age_hbm,
                   ac_hbm, na_hbm, rmax_hbm, rpos_hbm, rna_hbm, cnt_hbm,
                   sv, mv, lv, agev, nav, acv, tv, rmax, rpos, rna, rcnt,
                   posv, sem):
        wid = lax.axis_index("s") * _SC_CORES + lax.axis_index("c")
        base = wid * CH
        copies = [pltpu.async_copy(s_hbm.at[b, pl.ds(base, CH)], sv.at[b],
                                   sem) for b in range(batch)]
        copies.append(pltpu.async_copy(m_hbm, mv, sem))
        copies.append(pltpu.async_copy(l_hbm, lv, sem))
        copies.append(pltpu.async_copy(age_hbm.at[0, pl.ds(base, CH)],
                                       agev, sem))
        for c in copies:
            c.wait()

        mvec = [mv[b, pl.ds(0, _LANES)] for b in range(batch)]
        lvec = [lv[b, pl.ds(0, _LANES)] for b in range(batch)]

        @plsc.parallel_loop(0, NSL)
        def _(j):
            sl = pl.ds(j * _LANES, _LANES)
            imp = jnp.zeros((_LANES,), f32)
            ac = jnp.zeros((_LANES,), i32)
            for b in range(batch):
                p = jnp.exp(sv[b, sl] - mvec[b]) / lvec[b]
                imp = imp + p
                ac = ac + jnp.where(p > 0.01, 1, 0).astype(i32)
            na = agev[sl] + 1.0
            nav[sl] = na
            acv[sl] = ac
            tv[sl] = na + (1.0 - imp)

        rmax[...] = jnp.full((_LANES,), -jnp.inf, f32)
        rpos[...] = jnp.zeros((_LANES,), i32)
        rna[...] = jnp.zeros((_LANES,), f32)
        rcnt[...] = jnp.zeros((_LANES,), f32)
        posv[...] = lax.iota(i32, _LANES)

        @pl.loop(0, NSL)
        def _(j):
            sl = pl.ds(j * _LANES, _LANES)
            t = tv[sl]
            na = nav[sl]
            pv = posv[...]
            upd = t > rmax[...]
            rmax[...] = jnp.where(upd, t, rmax[...])
            rpos[...] = jnp.where(upd, pv, rpos[...])
            rna[...] = jnp.where(upd, na, rna[...])
            rcnt[...] = rcnt[...] + jnp.where(na > 0.0, 1.0, 0.0)
            posv[...] = pv + _LANES

        pltpu.sync_copy(acv, ac_hbm.at[0, pl.ds(base, CH)])
        pltpu.sync_copy(nav, na_hbm.at[0, pl.ds(base, CH)])
        pltpu.sync_copy(rmax, rmax_hbm.at[wid, pl.ds(0, _LANES)])
        pltpu.sync_copy(rpos, rpos_hbm.at[wid, pl.ds(0, _LANES)])
        pltpu.sync_copy(rna, rna_hbm.at[wid, pl.ds(0, _LANES)])
        pltpu.sync_copy(rcnt, cnt_hbm.at[wid, pl.ds(0, _LANES)])

    ac_row, na_row, rmax_a, rpos_a, rna_a, cnt_a = _sc_select(
        scores, mrow, lrow, age2)

    def _vstream(s_ref, m_ref, l_ref, v_ref, wo_ref, bo_ref,
                 nv_ref, y_ref, o_scr):
        i = pl.program_id(0)

        @pl.when(i == 0)
        def _():
            o_scr[...] = jnp.zeros(o_scr.shape, f32)

        m = m_ref[:, 0:1]
        l = l_ref[:, 0:1]
        p = jnp.exp(s_ref[...] - m) / l
        v = v_ref[...]
        nv_ref[...] = v
        o_scr[...] += jax.lax.dot_general(p, v, dn_nn,
                                          preferred_element_type=f32)

        @pl.when(i == NB - 1)
        def _():
            y_ref[...] = jax.lax.dot_general(
                o_scr[...], wo_ref[...], dn_nt,
                preferred_element_type=f32) + bo_ref[...]

    new_values2, out = pl.pallas_call(
        _vstream,
        grid=(NB,),
        in_specs=[pl.BlockSpec((batch, BM), lambda i: (0, i)),
                  pl.BlockSpec((batch, 128), lambda i: (0, 0)),
                  pl.BlockSpec((batch, 128), lambda i: (0, 0)),
                  pl.BlockSpec((BM, hidden), lambda i: (i, 0)),
                  pl.BlockSpec((hidden, hidden), lambda i: (0, 0)),
                  pl.BlockSpec((1, hidden), lambda i: (0, 0))],
        out_specs=[pl.BlockSpec((BM, hidden), lambda i: (i, 0)),
                   pl.BlockSpec((batch, hidden), lambda i: (0, 0))],
        out_shape=[jax.ShapeDtypeStruct((msize, hidden), f32),
                   jax.ShapeDtypeStruct((batch, hidden), f32)],
        scratch_shapes=[pltpu.VMEM((batch, hidden), f32)],
    )(scores, mrow, lrow, V2, Wo, bo.reshape(1, hidden))

    def _select(rmax_ref, rpos_ref, rna_ref, cnt_ref, idx_ref, usage_ref):
        sub = rmax_ref[:, 0:_LANES]
        pos = rpos_ref[:, 0:_LANES]
        rna = rna_ref[:, 0:_LANES]
        cnt = cnt_ref[:, 0:_LANES]
        rowi = jax.lax.broadcasted_iota(i32, sub.shape, 0)
        pabs = rowi * CH + pos
        gmax = jnp.max(sub)
        cand = jnp.where(sub == gmax, pabs, msize)
        gidx = jnp.min(cand)
        na_at = jnp.sum(jnp.where((sub == gmax) & (pabs == gidx), rna, 0.0))
        npos = jnp.sum(cnt)
        usage = (npos - (na_at > 0.0).astype(f32)) / msize
        idx_ref[...] = jnp.full(idx_ref.shape, gidx, i32)
        usage_ref[...] = jnp.full(usage_ref.shape, usage, f32)

    idx_out, usage_out = pl.pallas_call(
        _select,
        out_shape=[jax.ShapeDtypeStruct((1, 128), i32),
                   jax.ShapeDtypeStruct((1, 128), f32)],
    )(rmax_a, rpos_a, rna_a, cnt_a)

    idx1 = idx_out[0, 0:1]
    updk = kproj[0:1]
    updv = vproj[0:1]

    def _scatter(idx_ref, updk_ref, updv_ref, kin_ref, vin_ref, ain_ref,
                 kout_ref, vout_ref, aout_ref):
        row = idx_ref[0] % 8
        lane = idx_ref[0] % 128
        rowv = jax.lax.broadcasted_iota(i32, kin_ref.shape, 0)
        kout_ref[...] = jnp.where(rowv == row, updk_ref[...], kin_ref[...])
        vout_ref[...] = jnp.where(rowv == row, updv_ref[...], vin_ref[...])
        colv = jax.lax.broadcasted_iota(i32, ain_ref.shape, 1)
        aout_ref[...] = jnp.where(colv == lane, 0.0, ain_ref[...])

    grid_spec = pltpu.PrefetchScalarGridSpec(
        num_scalar_prefetch=1,
        grid=(1,),
        in_specs=[
            pl.BlockSpec((1, hidden), lambda i, idx: (0, 0)),
            pl.BlockSpec((1, hidden), lambda i, idx: (0, 0)),
            pl.BlockSpec((8, hidden), lambda i, idx: (idx[0] // 8, 0)),
            pl.BlockSpec((8, hidden), lambda i, idx: (idx[0] // 8, 0)),
            pl.BlockSpec((1, 128), lambda i, idx: (0, idx[0] // 128)),
        ],
        out_specs=[
            pl.BlockSpec((8, hidden), lambda i, idx: (idx[0] // 8, 0)),
            pl.BlockSpec((8, hidden), lambda i, idx: (idx[0] // 8, 0)),
            pl.BlockSpec((1, 128), lambda i, idx: (0, idx[0] // 128)),
        ],
    )
    nk_f, nv_f, na_f = pl.pallas_call(
        _scatter,
        grid_spec=grid_spec,
        out_shape=[jax.ShapeDtypeStruct((msize, hidden), f32),
                   jax.ShapeDtypeStruct((msize, hidden), f32),
                   jax.ShapeDtypeStruct((1, msize), f32)],
        input_output_aliases={3: 0, 4: 1, 5: 2},
    )(idx1, updk, updv, new_keys2, new_values2, na_row)

    output = out.reshape(batch, seq, hidden)
    access_counts = ac_row.reshape(heads, msize)
    max_scores = msarr[0, 0]
    memory_usage = usage_out[0, 0]
    new_keys = nk_f.reshape(heads, msize, hidden)
    new_values = nv_f.reshape(heads, msize, hidden)
    new_age = na_f.reshape(heads, msize)
    return (output, access_counts, max_scores, memory_usage,
            new_keys, new_values, new_age)
